# baseline (speedup 1.0000x reference)
import functools

import jax
import jax.numpy as jnp
from jax import lax
from jax.experimental import pallas as pl
from jax.experimental.pallas import tpu as pltpu
from jax.experimental.pallas import tpu_sc as plsc

_NC = 2
_NS = 16
_NW = _NC * _NS

_BATCH = 16384
_EMBED = 64
_IDX_CHUNK = 128
_B_PER_W = _BATCH // _NW
_CHUNKS_PER_W = _B_PER_W // _IDX_CHUNK


def _gather_body(user_idx_hbm, movie_idx_hbm, user_table_hbm, movie_table_hbm,
                 uvec_hbm, mvec_hbm, idx_u, idx_m, rows_u0, rows_u1, rows_m0,
                 rows_m1, sem_u0, sem_u1, sem_m0, sem_m1, sem_i):
  wid = lax.axis_index("s") * _NC + lax.axis_index("c")
  base = wid * _B_PER_W
  row0 = wid * _CHUNKS_PER_W
  ubufs = (rows_u0, rows_u1)
  mbufs = (rows_m0, rows_m1)
  usems = (sem_u0, sem_u1)
  msems = (sem_m0, sem_m1)
  cu = pltpu.async_copy(user_idx_hbm.at[pl.ds(row0, _CHUNKS_PER_W)], idx_u,
                        sem_i)
  cm = pltpu.async_copy(movie_idx_hbm.at[pl.ds(row0, _CHUNKS_PER_W)], idx_m,
                        sem_i)
  cu.wait()
  cm.wait()

  def start(j):
    b = j % 2
    cu = pltpu.async_copy(user_table_hbm.at[idx_u.at[j]], ubufs[b], usems[b])
    cm = pltpu.async_copy(movie_table_hbm.at[idx_m.at[j]], mbufs[b], msems[b])
    return cu, cm

  inflight = {0: start(0)}
  for j in range(_CHUNKS_PER_W):
    if j + 1 < _CHUNKS_PER_W:
      inflight[j + 1] = start(j + 1)
    cu, cm = inflight.pop(j)
    cu.wait()
    cm.wait()
    b = j % 2
    dst = pl.ds(base + j * _IDX_CHUNK, _IDX_CHUNK)
    pltpu.sync_copy(ubufs[b], uvec_hbm.at[dst])
    pltpu.sync_copy(mbufs[b], mvec_hbm.at[dst])


def _sc_gather_pairs(user_big, movie_big, user_table2, movie_table2):
  mesh = plsc.VectorSubcoreMesh(core_axis_name="c", subcore_axis_name="s")
  fn = pl.kernel(
      _gather_body,
      out_type=[
          jax.ShapeDtypeStruct((_BATCH, 2 * _EMBED), jnp.float32),
          jax.ShapeDtypeStruct((_BATCH, 2 * _EMBED), jnp.float32),
      ],
      mesh=mesh,
      compiler_params=pltpu.CompilerParams(use_tc_tiling_on_sc=True),
      scratch_types=[
          pltpu.VMEM((_CHUNKS_PER_W, _IDX_CHUNK), jnp.int32),
          pltpu.VMEM((_CHUNKS_PER_W, _IDX_CHUNK), jnp.int32),
          pltpu.VMEM((_IDX_CHUNK, 2 * _EMBED), jnp.float32),
          pltpu.VMEM((_IDX_CHUNK, 2 * _EMBED), jnp.float32),
          pltpu.VMEM((_IDX_CHUNK, 2 * _EMBED), jnp.float32),
          pltpu.VMEM((_IDX_CHUNK, 2 * _EMBED), jnp.float32),
          pltpu.SemaphoreType.DMA,
          pltpu.SemaphoreType.DMA,
          pltpu.SemaphoreType.DMA,
          pltpu.SemaphoreType.DMA,
          pltpu.SemaphoreType.DMA,
      ],
  )
  return fn(user_big, movie_big, user_table2, movie_table2)


_BLK = 2048


def _mlp_body(u2_ref, m2_ref, up_ref, mp_ref, w1u_ref, w1m_ref, b1_ref,
              w2t_ref, b2_ref, out_ref):
  up = up_ref[:]
  mp = mp_ref[:]
  u2 = u2_ref[:]
  m2 = m2_ref[:]
  u = u2[:, :_EMBED] * (1.0 - up) + u2[:, _EMBED:] * up
  m = m2[:, :_EMBED] * (1.0 - mp) + m2[:, _EMBED:] * mp
  h = (jnp.dot(u, w1u_ref[:], preferred_element_type=jnp.float32) +
       jnp.dot(m, w1m_ref[:], preferred_element_type=jnp.float32) +
       b1_ref[:])
  h = jnp.maximum(h, 0.0)
  out_ref[:] = (jnp.sum(h * w2t_ref[:], axis=1, keepdims=True) + b2_ref[:])


def _tc_mlp(u2, m2, upar, mpar, W1, b1, W2, b2):
  w1u = W1[:_EMBED]
  w1m = W1[_EMBED:]
  b1r = b1.reshape(1, 128)
  w2t = W2.reshape(1, 128)
  b2r = b2.reshape(1, 1)
  grid = (_BATCH // _BLK,)
  return pl.pallas_call(
      _mlp_body,
      grid=grid,
      in_specs=[
          pl.BlockSpec((_BLK, 2 * _EMBED), lambda i: (i, 0)),
          pl.BlockSpec((_BLK, 2 * _EMBED), lambda i: (i, 0)),
          pl.BlockSpec((_BLK, 1), lambda i: (i, 0)),
          pl.BlockSpec((_BLK, 1), lambda i: (i, 0)),
          pl.BlockSpec((_EMBED, 128), lambda i: (0, 0)),
          pl.BlockSpec((_EMBED, 128), lambda i: (0, 0)),
          pl.BlockSpec((1, 128), lambda i: (0, 0)),
          pl.BlockSpec((1, 128), lambda i: (0, 0)),
          pl.BlockSpec((1, 1), lambda i: (0, 0)),
      ],
      out_specs=pl.BlockSpec((_BLK, 1), lambda i: (i, 0)),
      out_shape=jax.ShapeDtypeStruct((_BATCH, 1), jnp.float32),
  )(u2, m2, upar, mpar, w1u, w1m, b1r, w2t, b2r)


@jax.jit
def _run(user, movie, user_table, movie_table, W1, b1, W2, b2):
  user_table2 = user_table.reshape(-1, 2 * _EMBED)
  movie_table2 = movie_table.reshape(-1, 2 * _EMBED)
  user_big = (user >> 1).reshape(_BATCH // _IDX_CHUNK, _IDX_CHUNK)
  movie_big = (movie >> 1).reshape(_BATCH // _IDX_CHUNK, _IDX_CHUNK)
  upar = (user & 1).astype(jnp.float32).reshape(_BATCH, 1)
  mpar = (movie & 1).astype(jnp.float32).reshape(_BATCH, 1)
  u2, m2 = _sc_gather_pairs(user_big, movie_big, user_table2, movie_table2)
  return _tc_mlp(u2, m2, upar, mpar, W1, b1, W2, b2)


def kernel(user, movie, user_table, movie_table, W1, b1, W2, b2):
  return _run(user, movie, user_table, movie_table, W1, b1, W2, b2)


# speedup vs baseline: 2.1607x; 2.1607x over previous
import functools

import jax
import jax.numpy as jnp
from jax import lax
from jax.experimental import pallas as pl
from jax.experimental.pallas import tpu as pltpu
from jax.experimental.pallas import tpu_sc as plsc

_NC = 2
_NS = 16
_NW = _NC * _NS

_BATCH = 16384
_EMBED = 64
_IDX_CHUNK = 128
_B_PER_W = _BATCH // _NW
_CHUNKS_PER_W = _B_PER_W // _IDX_CHUNK

_S_USER = 512000
_S_MOVIE = 51200
_TBH = 6400


def _tr_body(in1_ref, in2_ref, out_ref):
  out_ref[:] = jnp.concatenate([in1_ref[:].T, in2_ref[:].T], axis=1)


def _tc_transpose(table_t, s_half):
  grid = (s_half // _TBH,)
  off = s_half // _TBH
  last = (table_t.shape[1] - 1) // _TBH
  return pl.pallas_call(
      _tr_body,
      grid=grid,
      in_specs=[
          pl.BlockSpec((_EMBED, _TBH), lambda i: (0, i)),
          pl.BlockSpec((_EMBED, _TBH),
                       lambda i: (0, jnp.minimum(off + i, last))),
      ],
      out_specs=pl.BlockSpec((_TBH, 2 * _EMBED), lambda i: (i, 0)),
      out_shape=jax.ShapeDtypeStruct((s_half, 2 * _EMBED), jnp.float32),
  )(table_t, table_t)


def _gather_body(user_idx_hbm, movie_idx_hbm, user_table_hbm, movie_table_hbm,
                 uvec_hbm, mvec_hbm, idx_u, idx_m, rows_u0, rows_u1, rows_m0,
                 rows_m1, sem_u0, sem_u1, sem_m0, sem_m1, sem_i):
  wid = lax.axis_index("s") * _NC + lax.axis_index("c")
  base = wid * _B_PER_W
  row0 = wid * _CHUNKS_PER_W
  ubufs = (rows_u0, rows_u1)
  mbufs = (rows_m0, rows_m1)
  usems = (sem_u0, sem_u1)
  msems = (sem_m0, sem_m1)
  cu = pltpu.async_copy(user_idx_hbm.at[pl.ds(row0, _CHUNKS_PER_W)], idx_u,
                        sem_i)
  cm = pltpu.async_copy(movie_idx_hbm.at[pl.ds(row0, _CHUNKS_PER_W)], idx_m,
                        sem_i)
  cu.wait()
  cm.wait()

  def start(j):
    b = j % 2
    cu = pltpu.async_copy(user_table_hbm.at[idx_u.at[j]], ubufs[b], usems[b])
    cm = pltpu.async_copy(movie_table_hbm.at[idx_m.at[j]], mbufs[b], msems[b])
    return cu, cm

  inflight = {0: start(0)}
  for j in range(_CHUNKS_PER_W):
    if j + 1 < _CHUNKS_PER_W:
      inflight[j + 1] = start(j + 1)
    cu, cm = inflight.pop(j)
    cu.wait()
    cm.wait()
    b = j % 2
    dst = pl.ds(base + j * _IDX_CHUNK, _IDX_CHUNK)
    pltpu.sync_copy(ubufs[b], uvec_hbm.at[dst])
    pltpu.sync_copy(mbufs[b], mvec_hbm.at[dst])


def _sc_gather_pairs(user_big, movie_big, user_table2, movie_table2):
  mesh = plsc.VectorSubcoreMesh(core_axis_name="c", subcore_axis_name="s")
  fn = pl.kernel(
      _gather_body,
      out_type=[
          jax.ShapeDtypeStruct((_BATCH, 2 * _EMBED), jnp.float32),
          jax.ShapeDtypeStruct((_BATCH, 2 * _EMBED), jnp.float32),
      ],
      mesh=mesh,
      compiler_params=pltpu.CompilerParams(use_tc_tiling_on_sc=True),
      scratch_types=[
          pltpu.VMEM((_CHUNKS_PER_W, _IDX_CHUNK), jnp.int32),
          pltpu.VMEM((_CHUNKS_PER_W, _IDX_CHUNK), jnp.int32),
          pltpu.VMEM((_IDX_CHUNK, 2 * _EMBED), jnp.float32),
          pltpu.VMEM((_IDX_CHUNK, 2 * _EMBED), jnp.float32),
          pltpu.VMEM((_IDX_CHUNK, 2 * _EMBED), jnp.float32),
          pltpu.VMEM((_IDX_CHUNK, 2 * _EMBED), jnp.float32),
          pltpu.SemaphoreType.DMA,
          pltpu.SemaphoreType.DMA,
          pltpu.SemaphoreType.DMA,
          pltpu.SemaphoreType.DMA,
          pltpu.SemaphoreType.DMA,
      ],
  )
  return fn(user_big, movie_big, user_table2, movie_table2)


_BLK = 2048


def _mlp_body(u2_ref, m2_ref, up_ref, mp_ref, w1u_ref, w1m_ref, b1_ref,
              w2t_ref, b2_ref, out_ref):
  up = up_ref[:]
  mp = mp_ref[:]
  u2 = u2_ref[:]
  m2 = m2_ref[:]
  u = u2[:, :_EMBED] * (1.0 - up) + u2[:, _EMBED:] * up
  m = m2[:, :_EMBED] * (1.0 - mp) + m2[:, _EMBED:] * mp
  h = (jnp.dot(u, w1u_ref[:], preferred_element_type=jnp.float32) +
       jnp.dot(m, w1m_ref[:], preferred_element_type=jnp.float32) +
       b1_ref[:])
  h = jnp.maximum(h, 0.0)
  out_ref[:] = (jnp.sum(h * w2t_ref[:], axis=1, keepdims=True) + b2_ref[:])


def _tc_mlp(u2, m2, upar, mpar, W1, b1, W2, b2):
  w1u = W1[:_EMBED]
  w1m = W1[_EMBED:]
  b1r = b1.reshape(1, 128)
  w2t = W2.reshape(1, 128)
  b2r = b2.reshape(1, 1)
  grid = (_BATCH // _BLK,)
  return pl.pallas_call(
      _mlp_body,
      grid=grid,
      in_specs=[
          pl.BlockSpec((_BLK, 2 * _EMBED), lambda i: (i, 0)),
          pl.BlockSpec((_BLK, 2 * _EMBED), lambda i: (i, 0)),
          pl.BlockSpec((_BLK, 1), lambda i: (i, 0)),
          pl.BlockSpec((_BLK, 1), lambda i: (i, 0)),
          pl.BlockSpec((_EMBED, 128), lambda i: (0, 0)),
          pl.BlockSpec((_EMBED, 128), lambda i: (0, 0)),
          pl.BlockSpec((1, 128), lambda i: (0, 0)),
          pl.BlockSpec((1, 128), lambda i: (0, 0)),
          pl.BlockSpec((1, 1), lambda i: (0, 0)),
      ],
      out_specs=pl.BlockSpec((_BLK, 1), lambda i: (i, 0)),
      out_shape=jax.ShapeDtypeStruct((_BATCH, 1), jnp.float32),
  )(u2, m2, upar, mpar, w1u, w1m, b1r, w2t, b2r)


@jax.jit
def _run(user, movie, user_table, movie_table, W1, b1, W2, b2):
  user_table2 = _tc_transpose(user_table.T, _S_USER)
  movie_table2 = _tc_transpose(movie_table.T, _S_MOVIE)
  user_big = jnp.where(user < _S_USER, user,
                       user - _S_USER).reshape(_BATCH // _IDX_CHUNK,
                                               _IDX_CHUNK)
  movie_big = jnp.where(movie < _S_MOVIE, movie,
                        movie - _S_MOVIE).reshape(_BATCH // _IDX_CHUNK,
                                                  _IDX_CHUNK)
  upar = (user >= _S_USER).astype(jnp.float32).reshape(_BATCH, 1)
  mpar = (movie >= _S_MOVIE).astype(jnp.float32).reshape(_BATCH, 1)
  u2, m2 = _sc_gather_pairs(user_big, movie_big, user_table2, movie_table2)
  return _tc_mlp(u2, m2, upar, mpar, W1, b1, W2, b2)


def kernel(user, movie, user_table, movie_table, W1, b1, W2, b2):
  return _run(user, movie, user_table, movie_table, W1, b1, W2, b2)


# speedup vs baseline: 2.5592x; 1.1844x over previous
import functools

import jax
import jax.numpy as jnp
from jax import lax
from jax.experimental import pallas as pl
from jax.experimental.pallas import tpu as pltpu
from jax.experimental.pallas import tpu_sc as plsc

_NC = 2
_NS = 16
_NW = _NC * _NS

_BATCH = 16384
_EMBED = 64
_IDX_CHUNK = 128
_B_PER_W = _BATCH // _NW
_CHUNKS_PER_W = _B_PER_W // _IDX_CHUNK

_Q_USER = 256000
_Q_MOVIE = 25600
_TBH = 6400


def _pack_body(in1_ref, in2_ref, in3_ref, in4_ref, out_ref):
  def bf16_bits(x_t):
    xi = jax.lax.bitcast_convert_type(x_t, jnp.int32)
    return jax.lax.shift_right_logical(xi + jnp.int32(0x8000), 16)

  a = bf16_bits(in1_ref[:].T)
  b = bf16_bits(in2_ref[:].T)
  c = bf16_bits(in3_ref[:].T)
  d = bf16_bits(in4_ref[:].T)
  w_lo = a | jax.lax.shift_left(b, 16)
  w_hi = c | jax.lax.shift_left(d, 16)
  out_ref[:] = jnp.concatenate([w_lo, w_hi], axis=1)


def _tc_pack(table_t, q_quarter):
  grid = (q_quarter // _TBH,)
  off = q_quarter // _TBH
  last = (table_t.shape[1] - 1) // _TBH
  return pl.pallas_call(
      _pack_body,
      grid=grid,
      in_specs=[
          pl.BlockSpec((_EMBED, _TBH), lambda i: (0, i)),
          pl.BlockSpec((_EMBED, _TBH),
                       lambda i: (0, jnp.minimum(off + i, last))),
          pl.BlockSpec((_EMBED, _TBH),
                       lambda i: (0, jnp.minimum(2 * off + i, last))),
          pl.BlockSpec((_EMBED, _TBH),
                       lambda i: (0, jnp.minimum(3 * off + i, last))),
      ],
      out_specs=pl.BlockSpec((_TBH, 2 * _EMBED), lambda i: (i, 0)),
      out_shape=jax.ShapeDtypeStruct((q_quarter, 2 * _EMBED), jnp.int32),
  )(table_t, table_t, table_t, table_t)


def _gather_body(user_idx_hbm, movie_idx_hbm, user_table_hbm, movie_table_hbm,
                 uvec_hbm, mvec_hbm, idx_u, idx_m, rows_u0, rows_u1, rows_m0,
                 rows_m1, sem_u0, sem_u1, sem_m0, sem_m1, sem_i):
  wid = lax.axis_index("s") * _NC + lax.axis_index("c")
  base = wid * _B_PER_W
  row0 = wid * _CHUNKS_PER_W
  ubufs = (rows_u0, rows_u1)
  mbufs = (rows_m0, rows_m1)
  usems = (sem_u0, sem_u1)
  msems = (sem_m0, sem_m1)
  cu = pltpu.async_copy(user_idx_hbm.at[pl.ds(row0, _CHUNKS_PER_W)], idx_u,
                        sem_i)
  cm = pltpu.async_copy(movie_idx_hbm.at[pl.ds(row0, _CHUNKS_PER_W)], idx_m,
                        sem_i)
  cu.wait()
  cm.wait()

  def start(j):
    b = j % 2
    cu = pltpu.async_copy(user_table_hbm.at[idx_u.at[j]], ubufs[b], usems[b])
    cm = pltpu.async_copy(movie_table_hbm.at[idx_m.at[j]], mbufs[b], msems[b])
    return cu, cm

  inflight = {0: start(0)}
  for j in range(_CHUNKS_PER_W):
    if j + 1 < _CHUNKS_PER_W:
      inflight[j + 1] = start(j + 1)
    cu, cm = inflight.pop(j)
    cu.wait()
    cm.wait()
    b = j % 2
    dst = pl.ds(base + j * _IDX_CHUNK, _IDX_CHUNK)
    pltpu.sync_copy(ubufs[b], uvec_hbm.at[dst])
    pltpu.sync_copy(mbufs[b], mvec_hbm.at[dst])


def _sc_gather_quads(user_q, movie_q, user_table4, movie_table4):
  mesh = plsc.VectorSubcoreMesh(core_axis_name="c", subcore_axis_name="s")
  fn = pl.kernel(
      _gather_body,
      out_type=[
          jax.ShapeDtypeStruct((_BATCH, 2 * _EMBED), jnp.int32),
          jax.ShapeDtypeStruct((_BATCH, 2 * _EMBED), jnp.int32),
      ],
      mesh=mesh,
      compiler_params=pltpu.CompilerParams(use_tc_tiling_on_sc=True),
      scratch_types=[
          pltpu.VMEM((_CHUNKS_PER_W, _IDX_CHUNK), jnp.int32),
          pltpu.VMEM((_CHUNKS_PER_W, _IDX_CHUNK), jnp.int32),
          pltpu.VMEM((_IDX_CHUNK, 2 * _EMBED), jnp.int32),
          pltpu.VMEM((_IDX_CHUNK, 2 * _EMBED), jnp.int32),
          pltpu.VMEM((_IDX_CHUNK, 2 * _EMBED), jnp.int32),
          pltpu.VMEM((_IDX_CHUNK, 2 * _EMBED), jnp.int32),
          pltpu.SemaphoreType.DMA,
          pltpu.SemaphoreType.DMA,
          pltpu.SemaphoreType.DMA,
          pltpu.SemaphoreType.DMA,
          pltpu.SemaphoreType.DMA,
      ],
  )
  return fn(user_q, movie_q, user_table4, movie_table4)


_BLK = 2048


def _unpack(w4, hi_sel, odd_sel):
  w = w4[:, _EMBED:] * hi_sel + w4[:, :_EMBED] * (1 - hi_sel)
  lo = jax.lax.shift_left(w, 16)
  hi = w & jnp.int32(-65536)
  bits = hi * odd_sel + lo * (1 - odd_sel)
  return jax.lax.bitcast_convert_type(bits, jnp.float32)


def _mlp_body(u4_ref, m4_ref, uq_ref, mq_ref, w1u_ref, w1m_ref, b1_ref,
              w2t_ref, b2_ref, out_ref):
  uq = uq_ref[:]
  mq = mq_ref[:]
  u = _unpack(u4_ref[:], uq[:, 0:1], uq[:, 1:2])
  m = _unpack(m4_ref[:], mq[:, 0:1], mq[:, 1:2])
  h = (jnp.dot(u, w1u_ref[:], preferred_element_type=jnp.float32) +
       jnp.dot(m, w1m_ref[:], preferred_element_type=jnp.float32) +
       b1_ref[:])
  h = jnp.maximum(h, 0.0)
  out_ref[:] = (jnp.sum(h * w2t_ref[:], axis=1, keepdims=True) + b2_ref[:])


def _tc_mlp(u4, m4, uq, mq, W1, b1, W2, b2):
  w1u = W1[:_EMBED]
  w1m = W1[_EMBED:]
  b1r = b1.reshape(1, 128)
  w2t = W2.reshape(1, 128)
  b2r = b2.reshape(1, 1)
  grid = (_BATCH // _BLK,)
  return pl.pallas_call(
      _mlp_body,
      grid=grid,
      in_specs=[
          pl.BlockSpec((_BLK, 2 * _EMBED), lambda i: (i, 0)),
          pl.BlockSpec((_BLK, 2 * _EMBED), lambda i: (i, 0)),
          pl.BlockSpec((_BLK, 2), lambda i: (i, 0)),
          pl.BlockSpec((_BLK, 2), lambda i: (i, 0)),
          pl.BlockSpec((_EMBED, 128), lambda i: (0, 0)),
          pl.BlockSpec((_EMBED, 128), lambda i: (0, 0)),
          pl.BlockSpec((1, 128), lambda i: (0, 0)),
          pl.BlockSpec((1, 128), lambda i: (0, 0)),
          pl.BlockSpec((1, 1), lambda i: (0, 0)),
      ],
      out_specs=pl.BlockSpec((_BLK, 1), lambda i: (i, 0)),
      out_shape=jax.ShapeDtypeStruct((_BATCH, 1), jnp.float32),
  )(u4, m4, uq, mq, w1u, w1m, b1r, w2t, b2r)


@jax.jit
def _run(user, movie, user_table, movie_table, W1, b1, W2, b2):
  user_table4 = _tc_pack(user_table.T, _Q_USER)
  movie_table4 = _tc_pack(movie_table.T, _Q_MOVIE)
  uqr = user // _Q_USER
  mqr = movie // _Q_MOVIE
  user_q = (user - uqr * _Q_USER).reshape(_BATCH // _IDX_CHUNK, _IDX_CHUNK)
  movie_q = (movie - mqr * _Q_MOVIE).reshape(_BATCH // _IDX_CHUNK, _IDX_CHUNK)
  uq = jnp.stack([(uqr >= 2).astype(jnp.int32), (uqr & 1)], axis=1)
  mq = jnp.stack([(mqr >= 2).astype(jnp.int32), (mqr & 1)], axis=1)
  u4, m4 = _sc_gather_quads(user_q, movie_q, user_table4, movie_table4)
  return _tc_mlp(u4, m4, uq, mq, W1, b1, W2, b2)


def kernel(user, movie, user_table, movie_table, W1, b1, W2, b2):
  return _run(user, movie, user_table, movie_table, W1, b1, W2, b2)
